# Initial kernel scaffold; baseline (speedup 1.0000x reference)
import jax
import jax.numpy as jnp
from jax.experimental import pallas as pl


def kernel(idx, targets, token_embedding_table):
    raise NotImplementedError("write your pallas kernel here")


# speedup vs baseline: 1.1707x; 1.1707x over previous
import functools

import jax
import jax.numpy as jnp
from jax import lax
from jax.experimental import pallas as pl
from jax.experimental.pallas import tpu as pltpu
from jax.experimental.pallas import tpu_sc as plsc

VOCAB_SIZE = 1000
N_TOK = 1024 * 200

NUM_CORES = 2
NUM_SUBCORES = 16
LANES = 16
NW = NUM_CORES * NUM_SUBCORES
TOK_PER_W = N_TOK // NW
CHUNK = 32
NCHUNK = TOK_PER_W // CHUNK


def _logz_body(tab_ref, out_ref):
    x = tab_ref[...]
    m = jnp.max(x, axis=1)
    s = jnp.sum(jnp.exp(x - m[:, None]), axis=1)
    out_ref[...] = m + jnp.log(s)


def _compute_logz(table):
    return pl.pallas_call(
        _logz_body,
        out_shape=jax.ShapeDtypeStruct((VOCAB_SIZE,), jnp.float32),
    )(table)


def _sc_body(table_hbm, idx_hbm, tgt_hbm, logz_hbm, out_hbm, part_hbm,
             idx_v, tgt_v, logz_v, rows_v, acc_v, sem_g):
    wid = lax.axis_index("s") * NUM_CORES + lax.axis_index("c")
    base = wid * TOK_PER_W
    pltpu.sync_copy(idx_hbm.at[pl.ds(base, TOK_PER_W)], idx_v)
    pltpu.sync_copy(tgt_hbm.at[pl.ds(base, TOK_PER_W)], tgt_v)
    pltpu.sync_copy(logz_hbm, logz_v)

    def chunk_body(k, acc):
        off = pl.multiple_of(k * CHUNK, 8)
        pltpu.async_copy(table_hbm.at[idx_v.at[pl.ds(off, CHUNK)]],
                         rows_v, sem_g).wait()
        for s in range(CHUNK // LANES):
            lo = pl.multiple_of(off + s * LANES, 8)
            ids = idx_v[pl.ds(lo, LANES)]
            tgs = tgt_v[pl.ds(lo, LANES)]
            rid = lax.iota(jnp.int32, LANES) + s * LANES
            lz = plsc.load_gather(logz_v, [ids])
            tl = plsc.load_gather(rows_v, [rid, tgs])
            acc = acc + (lz - tl)
        pltpu.sync_copy(rows_v, out_hbm.at[pl.ds(base + off, CHUNK)])
        return acc

    acc = lax.fori_loop(0, NCHUNK, chunk_body, jnp.zeros((LANES,), jnp.float32))
    acc_v[...] = acc
    pltpu.sync_copy(acc_v, part_hbm.at[wid])


def _sc_gather(table, idx_f, tgt_f, logz):
    mesh = plsc.VectorSubcoreMesh(core_axis_name="c", subcore_axis_name="s")
    fn = functools.partial(
        pl.kernel,
        mesh=mesh,
        out_type=[
            jax.ShapeDtypeStruct((N_TOK, VOCAB_SIZE), jnp.float32),
            jax.ShapeDtypeStruct((NW, LANES), jnp.float32),
        ],
        scratch_types=[
            pltpu.VMEM((TOK_PER_W,), jnp.int32),
            pltpu.VMEM((TOK_PER_W,), jnp.int32),
            pltpu.VMEM((VOCAB_SIZE,), jnp.float32),
            pltpu.VMEM((CHUNK, VOCAB_SIZE), jnp.float32),
            pltpu.VMEM((LANES,), jnp.float32),
            pltpu.SemaphoreType.DMA,
        ],
        compiler_params=pltpu.CompilerParams(
            needs_layout_passes=False,
            use_tc_tiling_on_sc=False,
        ),
    )(_sc_body)
    return fn(table, idx_f, tgt_f, logz)


def _reduce_body(p_ref, out_ref):
    out_ref[...] = jnp.sum(p_ref[...]).reshape(1, 1) * (1.0 / N_TOK)


def _reduce_loss(part):
    return pl.pallas_call(
        _reduce_body,
        out_shape=jax.ShapeDtypeStruct((1, 1), jnp.float32),
    )(part)


def kernel(idx, targets, token_embedding_table):
    idx_f = idx.reshape(-1).astype(jnp.int32)
    tgt_f = targets.reshape(-1).astype(jnp.int32)
    logz = _compute_logz(token_embedding_table)
    logits, part = _sc_gather(token_embedding_table, idx_f, tgt_f, logz)
    loss = _reduce_loss(part)[0, 0]
    return (logits, loss)


# speedup vs baseline: 1.2505x; 1.0682x over previous
import functools

import jax
import jax.numpy as jnp
from jax import lax
from jax.experimental import pallas as pl
from jax.experimental.pallas import tpu as pltpu
from jax.experimental.pallas import tpu_sc as plsc

VOCAB_SIZE = 1000
N_TOK = 1024 * 200

NUM_CORES = 2
NUM_SUBCORES = 16
LANES = 16
NW = NUM_CORES * NUM_SUBCORES
TOK_PER_W = N_TOK // NW
CHUNK = 16
NCHUNK = TOK_PER_W // CHUNK
NBUF = 4


def _logz_body(tab_ref, out_ref):
    x = tab_ref[...]
    m = jnp.max(x, axis=1)
    s = jnp.sum(jnp.exp(x - m[:, None]), axis=1)
    out_ref[...] = m + jnp.log(s)


def _compute_logz(table):
    return pl.pallas_call(
        _logz_body,
        out_shape=jax.ShapeDtypeStruct((VOCAB_SIZE,), jnp.float32),
    )(table)


def _sc_body(table_hbm, idx_hbm, tgt_hbm, logz_hbm, out_hbm, part_hbm,
             idx_v, tgt_v, logz_v, r0, r1, r2, r3, acc_v,
             sg0, sg1, sg2, sg3, ss0, ss1, ss2, ss3):
    rows = (r0, r1, r2, r3)
    sem_g = (sg0, sg1, sg2, sg3)
    sem_s = (ss0, ss1, ss2, ss3)
    wid = lax.axis_index("s") * NUM_CORES + lax.axis_index("c")
    base = wid * TOK_PER_W
    pltpu.sync_copy(idx_hbm.at[pl.ds(base, TOK_PER_W)], idx_v)
    pltpu.sync_copy(tgt_hbm.at[pl.ds(base, TOK_PER_W)], tgt_v)
    pltpu.sync_copy(logz_hbm, logz_v)

    def start_gather(k, j):
        off = pl.multiple_of(k * CHUNK, 8)
        pltpu.async_copy(table_hbm.at[idx_v.at[pl.ds(off, CHUNK)]],
                         rows[j], sem_g[j])

    def wait_gather(j):
        pltpu.make_async_copy(table_hbm.at[idx_v.at[pl.ds(0, CHUNK)]],
                              rows[j], sem_g[j]).wait()

    def start_scatter(k, j):
        off = pl.multiple_of(k * CHUNK, 8)
        pltpu.async_copy(rows[j], out_hbm.at[pl.ds(base + off, CHUNK)],
                         sem_s[j])

    def wait_scatter(j):
        pltpu.make_async_copy(rows[j], out_hbm.at[pl.ds(base, CHUNK)],
                              sem_s[j]).wait()

    for j in range(NBUF - 1):
        start_gather(j, j)

    def group(g, acc):
        for j in range(NBUF):
            k = g * NBUF + j
            jj = (j + NBUF - 1) % NBUF

            @pl.when(jnp.logical_and(k >= 1, k + NBUF - 1 < NCHUNK))
            def _():
                wait_scatter(jj)
                start_gather(k + NBUF - 1, jj)

            @pl.when(k == 0)
            def _():
                start_gather(NBUF - 1, NBUF - 1)

            wait_gather(j)
            start_scatter(k, j)

            off = pl.multiple_of(k * CHUNK, 8)
            for s in range(CHUNK // LANES):
                lo = pl.multiple_of(off + s * LANES, 8)
                ids = idx_v[pl.ds(lo, LANES)]
                tgs = tgt_v[pl.ds(lo, LANES)]
                rid = lax.iota(jnp.int32, LANES) + s * LANES
                lz = plsc.load_gather(logz_v, [ids])
                tl = plsc.load_gather(rows[j], [rid, tgs])
                acc = acc + (lz - tl)
        return acc

    acc = lax.fori_loop(0, NCHUNK // NBUF, group,
                        jnp.zeros((LANES,), jnp.float32))
    for j in range(NBUF):
        wait_scatter(j)
    acc_v[...] = acc
    pltpu.sync_copy(acc_v, part_hbm.at[wid])


def _sc_gather(table, idx_f, tgt_f, logz):
    mesh = plsc.VectorSubcoreMesh(core_axis_name="c", subcore_axis_name="s")
    fn = functools.partial(
        pl.kernel,
        mesh=mesh,
        out_type=[
            jax.ShapeDtypeStruct((N_TOK, VOCAB_SIZE), jnp.float32),
            jax.ShapeDtypeStruct((NW, LANES), jnp.float32),
        ],
        scratch_types=[
            pltpu.VMEM((TOK_PER_W,), jnp.int32),
            pltpu.VMEM((TOK_PER_W,), jnp.int32),
            pltpu.VMEM((VOCAB_SIZE,), jnp.float32),
            pltpu.VMEM((CHUNK, VOCAB_SIZE), jnp.float32),
            pltpu.VMEM((CHUNK, VOCAB_SIZE), jnp.float32),
            pltpu.VMEM((CHUNK, VOCAB_SIZE), jnp.float32),
            pltpu.VMEM((CHUNK, VOCAB_SIZE), jnp.float32),
            pltpu.VMEM((LANES,), jnp.float32),
            pltpu.SemaphoreType.DMA,
            pltpu.SemaphoreType.DMA,
            pltpu.SemaphoreType.DMA,
            pltpu.SemaphoreType.DMA,
            pltpu.SemaphoreType.DMA,
            pltpu.SemaphoreType.DMA,
            pltpu.SemaphoreType.DMA,
            pltpu.SemaphoreType.DMA,
        ],
        compiler_params=pltpu.CompilerParams(
            needs_layout_passes=False,
            use_tc_tiling_on_sc=False,
        ),
    )(_sc_body)
    return fn(table, idx_f, tgt_f, logz)


def _reduce_body(p_ref, out_ref):
    out_ref[...] = jnp.sum(p_ref[...]).reshape(1, 1) * (1.0 / N_TOK)


def _reduce_loss(part):
    return pl.pallas_call(
        _reduce_body,
        out_shape=jax.ShapeDtypeStruct((1, 1), jnp.float32),
    )(part)


def kernel(idx, targets, token_embedding_table):
    idx_f = idx.reshape(-1).astype(jnp.int32)
    tgt_f = targets.reshape(-1).astype(jnp.int32)
    logz = _compute_logz(token_embedding_table)
    logits, part = _sc_gather(token_embedding_table, idx_f, tgt_f, logz)
    loss = _reduce_loss(part)[0, 0]
    return (logits, loss)


# speedup vs baseline: 1.8944x; 1.5149x over previous
import functools

import jax
import jax.numpy as jnp
from jax import lax
from jax.experimental import pallas as pl
from jax.experimental.pallas import tpu as pltpu
from jax.experimental.pallas import tpu_sc as plsc

VOCAB_SIZE = 1000
N_TOK = 1024 * 200

NUM_CORES = 2
NUM_SUBCORES = 16
LANES = 16
NW = NUM_CORES * NUM_SUBCORES
TOK_PER_W = N_TOK // NW
CHUNK = 16
NCHUNK = TOK_PER_W // CHUNK
NBUF = 4
VPAD = 1024


def _logz_body(tab_ref, out_ref):
    x = tab_ref[...]
    m = jnp.max(x, axis=1)
    s = jnp.sum(jnp.exp(x - m[:, None]), axis=1)
    out_ref[...] = m + jnp.log(s)


def _compute_logz(table):
    return pl.pallas_call(
        _logz_body,
        out_shape=jax.ShapeDtypeStruct((VOCAB_SIZE,), jnp.float32),
    )(table)


def _sc_body(table_hbm, idx_hbm, tgt_hbm, logz_hbm, flat_hbm,
             out_hbm, part_hbm,
             idx_v, tgt_v, acc_v, r0, r1, r2, r3, lz0, lz1, lz2, lz3,
             tl0, tl1, tl2, tl3,
             sg0, sg1, sg2, sg3, ss0, ss1, ss2, ss3,
             sz0, sz1, sz2, sz3, st0, st1, st2, st3):
    rows = (r0, r1, r2, r3)
    lzb = (lz0, lz1, lz2, lz3)
    tlb = (tl0, tl1, tl2, tl3)
    sem_g = (sg0, sg1, sg2, sg3)
    sem_s = (ss0, ss1, ss2, ss3)
    sem_z = (sz0, sz1, sz2, sz3)
    sem_t = (st0, st1, st2, st3)
    wid = lax.axis_index("s") * NUM_CORES + lax.axis_index("c")
    base = wid * TOK_PER_W
    pltpu.sync_copy(idx_hbm.at[pl.ds(base, TOK_PER_W)], idx_v)
    pltpu.sync_copy(tgt_hbm.at[pl.ds(base, TOK_PER_W)], tgt_v)

    def start_gather(k, j):
        off = pl.multiple_of(k * CHUNK, 8)
        ids = idx_v[pl.ds(off, LANES)]
        tgs = tgt_v[pl.ds(off, LANES)]
        pltpu.async_copy(table_hbm.at[idx_v.at[pl.ds(off, CHUNK)]],
                         rows[j], sem_g[j])
        pltpu.async_copy(logz_hbm.at[ids], lzb[j], sem_z[j])
        pltpu.async_copy(flat_hbm.at[ids * VOCAB_SIZE + tgs], tlb[j],
                         sem_t[j])

    def wait_gather(j):
        pltpu.make_async_copy(table_hbm.at[idx_v.at[pl.ds(0, CHUNK)]],
                              rows[j], sem_g[j]).wait()

    def start_scatter(k, j):
        off = pl.multiple_of(k * CHUNK, 8)
        pltpu.async_copy(rows[j], out_hbm.at[pl.ds(base + off, CHUNK)],
                         sem_s[j])

    def wait_scatter(j):
        pltpu.make_async_copy(rows[j], out_hbm.at[pl.ds(base, CHUNK)],
                              sem_s[j]).wait()

    def wait_aux(j):
        zero = jnp.zeros((LANES,), jnp.int32)
        pltpu.make_async_copy(logz_hbm.at[zero], lzb[j], sem_z[j]).wait()
        pltpu.make_async_copy(flat_hbm.at[zero], tlb[j], sem_t[j]).wait()

    for j in range(NBUF - 1):
        start_gather(j, j)

    def group(g, acc):
        for j in range(NBUF):
            k = g * NBUF + j
            jj = (j + NBUF - 1) % NBUF

            @pl.when(jnp.logical_and(k >= 1, k + NBUF - 1 < NCHUNK))
            def _():
                wait_scatter(jj)
                start_gather(k + NBUF - 1, jj)

            @pl.when(k == 0)
            def _():
                start_gather(NBUF - 1, NBUF - 1)

            wait_gather(j)
            start_scatter(k, j)

            wait_aux(j)
            acc = acc + (lzb[j][...] - tlb[j][...])
        return acc

    acc = lax.fori_loop(0, NCHUNK // NBUF, group,
                        jnp.zeros((LANES,), jnp.float32))
    for j in range(NBUF):
        wait_scatter(j)
    acc_v[...] = acc
    pltpu.sync_copy(acc_v, part_hbm.at[pl.ds(wid * LANES, LANES)])


def _sc_gather(table, idx_f, tgt_f, logz, flat):
    mesh = plsc.VectorSubcoreMesh(core_axis_name="c", subcore_axis_name="s")
    fn = functools.partial(
        pl.kernel,
        mesh=mesh,
        out_type=[
            jax.ShapeDtypeStruct((N_TOK, VPAD), jnp.float32),
            jax.ShapeDtypeStruct((NW * LANES,), jnp.float32),
        ],
        scratch_types=[
            pltpu.VMEM((TOK_PER_W,), jnp.int32),
            pltpu.VMEM((TOK_PER_W,), jnp.int32),
            pltpu.VMEM((LANES,), jnp.float32),
            pltpu.VMEM((CHUNK, VPAD), jnp.float32),
            pltpu.VMEM((CHUNK, VPAD), jnp.float32),
            pltpu.VMEM((CHUNK, VPAD), jnp.float32),
            pltpu.VMEM((CHUNK, VPAD), jnp.float32),
            pltpu.VMEM((LANES,), jnp.float32),
            pltpu.VMEM((LANES,), jnp.float32),
            pltpu.VMEM((LANES,), jnp.float32),
            pltpu.VMEM((LANES,), jnp.float32),
            pltpu.VMEM((LANES,), jnp.float32),
            pltpu.VMEM((LANES,), jnp.float32),
            pltpu.VMEM((LANES,), jnp.float32),
            pltpu.VMEM((LANES,), jnp.float32),
        ] + [pltpu.SemaphoreType.DMA] * 16,
        compiler_params=pltpu.CompilerParams(
            needs_layout_passes=False,
            use_tc_tiling_on_sc=True,
        ),
    )(_sc_body)
    return fn(table, idx_f, tgt_f, logz, flat)


def _reduce_body(p_ref, out_ref):
    out_ref[...] = jnp.sum(p_ref[...]).reshape(1, 1) * (1.0 / N_TOK)


def _reduce_loss(part):
    return pl.pallas_call(
        _reduce_body,
        out_shape=jax.ShapeDtypeStruct((1, 1), jnp.float32),
    )(part)


def kernel(idx, targets, token_embedding_table):
    idx_f = idx.reshape(-1).astype(jnp.int32)
    tgt_f = targets.reshape(-1).astype(jnp.int32)
    logz = _compute_logz(token_embedding_table)
    flat = token_embedding_table.reshape(-1)
    table_pad = jnp.pad(token_embedding_table,
                        ((0, 0), (0, VPAD - VOCAB_SIZE)))
    out_pad, part = _sc_gather(table_pad, idx_f, tgt_f, logz, flat)
    loss = _reduce_loss(part)[0, 0]
    return (out_pad[:, :VOCAB_SIZE], loss)


# speedup vs baseline: 1.9217x; 1.0144x over previous
import functools

import jax
import jax.numpy as jnp
from jax import lax
from jax.experimental import pallas as pl
from jax.experimental.pallas import tpu as pltpu
from jax.experimental.pallas import tpu_sc as plsc

VOCAB_SIZE = 1000
N_TOK = 1024 * 200

NUM_CORES = 2
NUM_SUBCORES = 16
LANES = 16
NW = NUM_CORES * NUM_SUBCORES
TOK_PER_W = N_TOK // NW
CHUNK = 32
NCHUNK = TOK_PER_W // CHUNK
NBUF = 3
VPAD = 1024


def _logz_body(tab_ref, out_ref):
    x = tab_ref[...]
    m = jnp.max(x, axis=1)
    s = jnp.sum(jnp.exp(x - m[:, None]), axis=1)
    out_ref[...] = m + jnp.log(s)


def _compute_logz(table):
    return pl.pallas_call(
        _logz_body,
        out_shape=jax.ShapeDtypeStruct((VOCAB_SIZE,), jnp.float32),
    )(table)


def _sc_body(table_hbm, idx_hbm, tgt_hbm, logz_hbm, flat_hbm,
             out_hbm, part_hbm,
             idx_v, tgt_v, lz_v, tl_v, acc_v, r0, r1, r2,
             sem_z, sem_t, sg0, sg1, sg2,
             ss0, ss1, ss2):
    rows = (r0, r1, r2)
    sem_g = (sg0, sg1, sg2)
    sem_s = (ss0, ss1, ss2)
    wid = lax.axis_index("s") * NUM_CORES + lax.axis_index("c")
    base = wid * TOK_PER_W
    pltpu.sync_copy(idx_hbm.at[pl.ds(base, TOK_PER_W)], idx_v)
    pltpu.sync_copy(tgt_hbm.at[pl.ds(base, TOK_PER_W)], tgt_v)

    def flat_body(i, carry):
        off = pl.multiple_of(i * LANES, 8)
        tgt_v[pl.ds(off, LANES)] = (idx_v[pl.ds(off, LANES)] * VOCAB_SIZE
                                    + tgt_v[pl.ds(off, LANES)])
        return carry
    lax.fori_loop(0, TOK_PER_W // LANES, flat_body, 0)

    pltpu.async_copy(logz_hbm.at[idx_v], lz_v, sem_z)
    pltpu.async_copy(flat_hbm.at[tgt_v], tl_v, sem_t)

    def start_gather(k, j):
        off = pl.multiple_of(k * CHUNK, 8)
        pltpu.async_copy(table_hbm.at[idx_v.at[pl.ds(off, CHUNK)]],
                         rows[j], sem_g[j])

    def wait_gather(j):
        pltpu.make_async_copy(table_hbm.at[idx_v.at[pl.ds(0, CHUNK)]],
                              rows[j], sem_g[j]).wait()

    def start_scatter(k, j):
        off = pl.multiple_of(k * CHUNK, 8)
        pltpu.async_copy(rows[j], out_hbm.at[pl.ds(base + off, CHUNK)],
                         sem_s[j])

    def wait_scatter(j):
        pltpu.make_async_copy(rows[j], out_hbm.at[pl.ds(base, CHUNK)],
                              sem_s[j]).wait()

    for j in range(NBUF - 1):
        start_gather(j, j)

    def group(g, carry):
        for j in range(NBUF):
            k = g * NBUF + j
            jj = (j + NBUF - 1) % NBUF

            @pl.when(jnp.logical_and(k >= 1, k + NBUF - 1 < NCHUNK))
            def _():
                wait_scatter(jj)
                start_gather(k + NBUF - 1, jj)

            @pl.when(k == 0)
            def _():
                start_gather(NBUF - 1, NBUF - 1)

            wait_gather(j)
            start_scatter(k, j)
        return carry

    lax.fori_loop(0, NCHUNK // NBUF, group, 0)
    for k in range((NCHUNK // NBUF) * NBUF, NCHUNK):
        j = k % NBUF
        wait_gather(j)
        start_scatter(k, j)
    for j in range(NBUF):
        wait_scatter(j)

    pltpu.make_async_copy(logz_hbm.at[idx_v], lz_v, sem_z).wait()
    pltpu.make_async_copy(flat_hbm.at[tgt_v], tl_v, sem_t).wait()

    def acc_body(i, acc):
        off = pl.multiple_of(i * LANES, 8)
        return acc + (lz_v[pl.ds(off, LANES)] - tl_v[pl.ds(off, LANES)])
    acc = lax.fori_loop(0, TOK_PER_W // LANES, acc_body,
                        jnp.zeros((LANES,), jnp.float32))
    acc_v[...] = acc
    pltpu.sync_copy(acc_v, part_hbm.at[pl.ds(wid * LANES, LANES)])


def _sc_gather(table, idx_f, tgt_f, logz, flat):
    mesh = plsc.VectorSubcoreMesh(core_axis_name="c", subcore_axis_name="s")
    fn = functools.partial(
        pl.kernel,
        mesh=mesh,
        out_type=[
            jax.ShapeDtypeStruct((N_TOK, VPAD), jnp.float32),
            jax.ShapeDtypeStruct((NW * LANES,), jnp.float32),
        ],
        scratch_types=[
            pltpu.VMEM((TOK_PER_W,), jnp.int32),
            pltpu.VMEM((TOK_PER_W,), jnp.int32),
            pltpu.VMEM((TOK_PER_W,), jnp.float32),
            pltpu.VMEM((TOK_PER_W,), jnp.float32),
            pltpu.VMEM((LANES,), jnp.float32),
        ] + [pltpu.VMEM((CHUNK, VPAD), jnp.float32)] * NBUF
          + [pltpu.SemaphoreType.DMA] * (2 + 2 * NBUF),
        compiler_params=pltpu.CompilerParams(
            needs_layout_passes=False,
            use_tc_tiling_on_sc=True,
        ),
    )(_sc_body)
    return fn(table, idx_f, tgt_f, logz, flat)


def _reduce_body(p_ref, out_ref):
    out_ref[...] = jnp.sum(p_ref[...]).reshape(1, 1) * (1.0 / N_TOK)


def _reduce_loss(part):
    return pl.pallas_call(
        _reduce_body,
        out_shape=jax.ShapeDtypeStruct((1, 1), jnp.float32),
    )(part)


def kernel(idx, targets, token_embedding_table):
    idx_f = idx.reshape(-1).astype(jnp.int32)
    tgt_f = targets.reshape(-1).astype(jnp.int32)
    logz = _compute_logz(token_embedding_table)
    flat = token_embedding_table.reshape(-1)
    table_pad = jnp.pad(token_embedding_table,
                        ((0, 0), (0, VPAD - VOCAB_SIZE)))
    out_pad, part = _sc_gather(table_pad, idx_f, tgt_f, logz, flat)
    loss = _reduce_loss(part)[0, 0]
    return (out_pad[:, :VOCAB_SIZE], loss)
